# baseline (speedup 1.0000x reference)
import jax
import jax.numpy as jnp
from jax import lax
from jax.experimental import pallas as pl
from jax.experimental.pallas import tpu as pltpu

_H = 4096
_E = 64
_N = 16384
_CAP = 512
_TB = 512


def _logits_body(x_ref, w_ref, out_ref):
    out_ref[...] = lax.dot_general(
        x_ref[...], w_ref[...],
        dimension_numbers=(((1,), (1,)), ((), ())),
        preferred_element_type=jnp.float32,
    )


def _router_logits(x_flat, w):
    grid = _N // _TB
    return pl.pallas_call(
        _logits_body,
        grid=(grid,),
        in_specs=[
            pl.BlockSpec((_TB, _H), lambda i: (i, 0)),
            pl.BlockSpec((_E, _H), lambda i: (0, 0)),
        ],
        out_specs=pl.BlockSpec((_TB, _E), lambda i: (i, 0)),
        out_shape=jax.ShapeDtypeStruct((_N, _E), jnp.float32),
    )(x_flat, w)


def kernel(hidden_states, W):
    b, s, h = hidden_states.shape
    x_flat = hidden_states.reshape(-1, h)
    logits = _router_logits(x_flat, W)
    weights = jax.nn.softmax(logits, axis=0)
    top_w, top_i = lax.top_k(weights.T, _CAP)
    return top_w.T.astype(hidden_states.dtype), top_i.T


# speedup vs baseline: 2.9345x; 2.9345x over previous
import functools

import jax
import jax.numpy as jnp
import numpy as np
from jax import lax
from jax.experimental import pallas as pl
from jax.experimental.pallas import tpu as pltpu
from jax.experimental.pallas import tpu_sc as plsc

_H = 4096
_E = 64
_N = 16384
_CAP = 512
_TB = 512

_NVREG = _N // 16
_CANDBUF = 4112
_INT_MIN = np.int32(-2147483648)
_IDX_SENT = np.int32(2147483647)


def _logits_body(x_ref, w_ref, p_ref, acc_ref):
    i = pl.program_id(0)
    nblk = _N // _TB
    r = lax.dot_general(
        x_ref[...], w_ref[...],
        dimension_numbers=(((1,), (1,)), ((), ())),
        preferred_element_type=jnp.float32,
    )
    acc_ref[i] = r.T

    @pl.when(i == nblk - 1)
    def _():
        m = jnp.max(acc_ref[0], axis=1, keepdims=True)
        for k in range(1, nblk):
            m = jnp.maximum(m, jnp.max(acc_ref[k], axis=1, keepdims=True))
        for k in range(nblk):
            p_ref[:, k * _TB:(k + 1) * _TB] = jnp.exp(acc_ref[k] - m)


def _p_rows(x_flat, w):
    grid = _N // _TB
    return pl.pallas_call(
        _logits_body,
        grid=(grid,),
        in_specs=[
            pl.BlockSpec((_TB, _H), lambda i: (i, 0)),
            pl.BlockSpec((_E, _H), lambda i: (0, 0)),
        ],
        out_specs=pl.BlockSpec((_E, _N), lambda i: (0, 0)),
        out_shape=jax.ShapeDtypeStruct((_E, _N), jnp.float32),
        scratch_shapes=[pltpu.VMEM((_N // _TB, _E, _TB), jnp.float32)],
    )(x_flat, w)


def _iota16():
    return lax.iota(jnp.int32, 16)


def _sortable(bits):
    return bits ^ (jnp.right_shift(bits, 31) & np.int32(0x7FFFFFFF))


def _scan_bins(hist, nbins, target):
    def read(b):
        return jnp.sum(hist[pl.ds(b * 16, 16)])

    def cond(state):
        b, g, c = state
        return (g + c) < target

    def body(state):
        b, g, c = state
        b2 = b - 1
        return (b2, g + c, read(b2))

    b0 = jnp.int32(nbins - 1)
    return lax.while_loop(cond, body, (b0, jnp.int32(0), read(b0)))


def _zero_hist(hist, nbins):
    zeros = jnp.zeros((16,), jnp.int32)

    def zbody(z, _):
        hist[pl.ds(z * 16, 16)] = zeros
        return 0

    lax.fori_loop(0, nbins, zbody, 0)


def _topk_body(lt_hbm, ow_hbm, oi_hbm,
               row, keys, hist, ck0, ci0, ck1, ci1, sk0, si0, sk1, si1, ow):
    info = plsc.get_sparse_core_info()
    nc = info.num_cores
    wid = lax.axis_index("s") * nc + lax.axis_index("c")
    iota = _iota16()
    ones = jnp.ones((16,), jnp.int32)

    def per_expert(j, _):
        e = wid * 2 + j
        pltpu.sync_copy(lt_hbm.at[e], row)

        _zero_hist(hist, 512)

        def p1(i, se):
            x = row[pl.ds(i * 16, 16)]
            se = se + x
            bits = lax.bitcast_convert_type(x, jnp.int32)
            key = _sortable(bits)
            keys[pl.ds(i * 16, 16)] = key
            bk = key ^ _INT_MIN
            b0 = jnp.right_shift(bk, 23) & np.int32(0x1FF)
            plsc.addupdate_scatter(hist, [b0 * 16 + iota], ones)
            return se

        se = lax.fori_loop(0, _NVREG, p1, jnp.zeros((16,), jnp.float32), unroll=8)
        ssum = jnp.sum(se)

        bin0, g0, c0 = _scan_bins(hist, 512, jnp.int32(_CAP))
        rem = jnp.int32(_CAP) - g0

        def p2(i, carry):
            od, oc = carry
            key = keys[pl.ds(i * 16, 16)]
            idx = i * 16 + iota
            bk = key ^ _INT_MIN
            b0 = jnp.right_shift(bk, 23) & np.int32(0x1FF)
            mhi = b0 > bin0
            meq = b0 == bin0
            chi = plsc.cumsum(mhi.astype(jnp.int32))
            ceq = plsc.cumsum(meq.astype(jnp.int32))
            phi = od + chi - 1
            peq = oc + ceq - 1
            plsc.store_scatter(sk0, [phi], key, mask=mhi)
            plsc.store_scatter(si0, [phi], idx, mask=mhi)
            plsc.store_scatter(ck0, [peq], key, mask=meq)
            plsc.store_scatter(ci0, [peq], idx, mask=meq)
            return (od + jnp.sum(mhi.astype(jnp.int32)),
                    oc + jnp.sum(meq.astype(jnp.int32)))

        od, oc = lax.fori_loop(0, _NVREG, p2, (jnp.int32(0), jnp.int32(0)), unroll=4)

        plsc.store_scatter(ck0, [oc + iota], jnp.full((16,), _INT_MIN))
        plsc.store_scatter(ci0, [oc + iota], jnp.full((16,), _IDX_SENT))

        def refine(src_k, src_i, dst_k, dst_i, m, shift, mask, nbins, od, rem):
            _zero_hist(hist, nbins)
            nv = (jnp.minimum(m, _CANDBUF - 16) + 15) // 16

            def hbody(i, _):
                key = src_k[pl.ds(i * 16, 16)]
                bk = key ^ _INT_MIN
                b = jnp.right_shift(bk, shift) & np.int32(mask)
                plsc.addupdate_scatter(hist, [b * 16 + iota], ones)
                return 0

            lax.fori_loop(0, nv, hbody, 0)
            bl, gl, cl = _scan_bins(hist, nbins, rem)

            def cbody(i, carry):
                odc, occ = carry
                key = src_k[pl.ds(i * 16, 16)]
                idx = src_i[pl.ds(i * 16, 16)]
                bk = key ^ _INT_MIN
                b = jnp.right_shift(bk, shift) & np.int32(mask)
                mhi = b > bl
                meq = b == bl
                chi = plsc.cumsum(mhi.astype(jnp.int32))
                ceq = plsc.cumsum(meq.astype(jnp.int32))
                phi = odc + chi - 1
                peq = occ + ceq - 1
                plsc.store_scatter(sk0, [phi], key, mask=mhi)
                plsc.store_scatter(si0, [phi], idx, mask=mhi)
                plsc.store_scatter(dst_k, [peq], key, mask=meq)
                plsc.store_scatter(dst_i, [peq], idx, mask=meq)
                return (odc + jnp.sum(mhi.astype(jnp.int32)),
                        occ + jnp.sum(meq.astype(jnp.int32)))

            od2, oc2 = lax.fori_loop(0, nv, cbody, (od, jnp.int32(0)))
            plsc.store_scatter(dst_k, [oc2 + iota], jnp.full((16,), _INT_MIN))
            plsc.store_scatter(dst_i, [oc2 + iota], jnp.full((16,), _IDX_SENT))
            return oc2, od2, rem - gl

        m, od, rem = refine(ck0, ci0, ck1, ci1, c0, 15, 0xFF, 256, od, rem)
        m, od, rem = refine(ck1, ci1, ck0, ci0, m, 7, 0xFF, 256, od, rem)
        m, od, rem = refine(ck0, ci0, ck1, ci1, m, 0, 0x7F, 128, od, rem)

        nv_t = (jnp.minimum(m, _CANDBUF - 16) + 15) // 16

        def tbody(i, _):
            lanepos = i * 16 + iota
            tm = lanepos < rem
            key = ck1[pl.ds(i * 16, 16)]
            idx = ci1[pl.ds(i * 16, 16)]
            plsc.store_scatter(sk0, [od + lanepos], key, mask=tm)
            plsc.store_scatter(si0, [od + lanepos], idx, mask=tm)
            return 0

        lax.fori_loop(0, nv_t, tbody, 0)

        bufs = ((sk0, si0), (sk1, si1))
        parity = 0
        ksz = 2
        while ksz <= _CAP:
            jj = ksz // 2
            while jj >= 1:
                src_k, src_i = bufs[parity]
                dst_k, dst_i = bufs[1 - parity]

                def stage(v, _, jj=jj, ksz=ksz, src_k=src_k, src_i=src_i,
                          dst_k=dst_k, dst_i=dst_i):
                    gi = v * 16 + iota
                    ks = src_k[pl.ds(v * 16, 16)]
                    is_ = src_i[pl.ds(v * 16, 16)]
                    if jj >= 16:
                        pb = (v * 16) ^ jj
                        kp = src_k[pl.ds(pb, 16)]
                        ip = src_i[pl.ds(pb, 16)]
                    else:
                        pi = gi ^ jj
                        kp = plsc.load_gather(src_k, [pi])
                        ip = plsc.load_gather(src_i, [pi])
                    d_desc = (gi & ksz) == 0
                    is_lower = (gi & jj) == 0
                    sel = d_desc == is_lower
                    cmp = (ks > kp) | ((ks == kp) & (is_ < ip))
                    keep = cmp == sel
                    dst_k[pl.ds(v * 16, 16)] = jnp.where(keep, ks, kp)
                    dst_i[pl.ds(v * 16, 16)] = jnp.where(keep, is_, ip)
                    return 0

                lax.fori_loop(0, _CAP // 16, stage, 0, unroll=4)
                parity = 1 - parity
                jj //= 2
            ksz *= 2

        fin_k, fin_i = bufs[parity]

        inv_v = jnp.full((16,), 1.0, jnp.float32) / lax.broadcast_in_dim(
            ssum, (16,), ())

        def obody(v, _):
            k = fin_k[pl.ds(v * 16, 16)]
            bits = _sortable(k)
            x = lax.bitcast_convert_type(bits, jnp.float32)
            ow[pl.ds(v * 16, 16)] = x * inv_v
            return 0

        lax.fori_loop(0, _CAP // 16, obody, 0)

        pltpu.sync_copy(ow, ow_hbm.at[e])
        pltpu.sync_copy(fin_i, oi_hbm.at[e])
        return 0

    lax.fori_loop(0, 2, per_expert, 0)


def _topk_sc(logits_t):
    mesh = plsc.VectorSubcoreMesh(core_axis_name="c", subcore_axis_name="s")
    fn = functools.partial(
        pl.kernel,
        mesh=mesh,
        compiler_params=pltpu.CompilerParams(needs_layout_passes=False),
        out_type=[
            jax.ShapeDtypeStruct((_E, _CAP), jnp.float32),
            jax.ShapeDtypeStruct((_E, _CAP), jnp.int32),
        ],
        scratch_types=[
            pltpu.VMEM((_N,), jnp.float32),
            pltpu.VMEM((_N,), jnp.int32),
            pltpu.VMEM((512 * 16,), jnp.int32),
            pltpu.VMEM((_CANDBUF,), jnp.int32),
            pltpu.VMEM((_CANDBUF,), jnp.int32),
            pltpu.VMEM((_CANDBUF,), jnp.int32),
            pltpu.VMEM((_CANDBUF,), jnp.int32),
            pltpu.VMEM((_CAP,), jnp.int32),
            pltpu.VMEM((_CAP,), jnp.int32),
            pltpu.VMEM((_CAP,), jnp.int32),
            pltpu.VMEM((_CAP,), jnp.int32),
            pltpu.VMEM((_CAP,), jnp.float32),
        ],
    )(_topk_body)
    return fn(logits_t)


def kernel(hidden_states, W):
    b, s, h = hidden_states.shape
    x_flat = hidden_states.reshape(-1, h)
    p_t = _p_rows(x_flat, W)
    w_t, i_t = _topk_sc(p_t)
    return w_t.T.astype(hidden_states.dtype), i_t.T


# speedup vs baseline: 2.9395x; 1.0017x over previous
import functools

import jax
import jax.numpy as jnp
import numpy as np
from jax import lax
from jax.experimental import pallas as pl
from jax.experimental.pallas import tpu as pltpu
from jax.experimental.pallas import tpu_sc as plsc

_H = 4096
_E = 64
_N = 16384
_CAP = 512
_TB = 512

_NVREG = _N // 16
_CANDBUF = 4112
_INT_MIN = np.int32(-2147483648)
_IDX_SENT = np.int32(2147483647)


def _logits_body(x_ref, w_ref, p_ref, acc_ref):
    i = pl.program_id(0)
    nblk = _N // _TB
    r = lax.dot_general(
        x_ref[...], w_ref[...],
        dimension_numbers=(((1,), (1,)), ((), ())),
        preferred_element_type=jnp.float32,
    )
    acc_ref[i] = r.T

    @pl.when(i == nblk - 1)
    def _():
        m = jnp.max(acc_ref[0], axis=1, keepdims=True)
        for k in range(1, nblk):
            m = jnp.maximum(m, jnp.max(acc_ref[k], axis=1, keepdims=True))
        for k in range(nblk):
            p_ref[:, k * _TB:(k + 1) * _TB] = jnp.exp(acc_ref[k] - m)


def _p_rows(x_flat, w):
    grid = _N // _TB
    return pl.pallas_call(
        _logits_body,
        grid=(grid,),
        in_specs=[
            pl.BlockSpec((_TB, _H), lambda i: (i, 0)),
            pl.BlockSpec((_E, _H), lambda i: (0, 0)),
        ],
        out_specs=pl.BlockSpec((_E, _N), lambda i: (0, 0)),
        out_shape=jax.ShapeDtypeStruct((_E, _N), jnp.float32),
        scratch_shapes=[pltpu.VMEM((_N // _TB, _E, _TB), jnp.float32)],
    )(x_flat, w)


def _iota16():
    return lax.iota(jnp.int32, 16)


def _sortable(bits):
    return bits ^ (jnp.right_shift(bits, 31) & np.int32(0x7FFFFFFF))


def _scan_bins(hist, nbins, target):
    def read(b):
        return jnp.sum(hist[pl.ds(b * 16, 16)])

    def cond(state):
        b, g, c = state
        return (g + c) < target

    def body(state):
        b, g, c = state
        b2 = b - 1
        return (b2, g + c, read(b2))

    b0 = jnp.int32(nbins - 1)
    return lax.while_loop(cond, body, (b0, jnp.int32(0), read(b0)))


def _zero_hist(hist, nbins):
    zeros = jnp.zeros((16,), jnp.int32)

    def zbody(z, _):
        hist[pl.ds(z * 16, 16)] = zeros
        return 0

    lax.fori_loop(0, nbins, zbody, 0)


def _topk_body(lt_hbm, ow_hbm, oi_hbm,
               row, keys, hist, ck0, ci0, ck1, ci1, sk0, si0, sk1, si1, ow):
    info = plsc.get_sparse_core_info()
    nc = info.num_cores
    wid = lax.axis_index("s") * nc + lax.axis_index("c")
    iota = _iota16()
    ones = jnp.ones((16,), jnp.int32)

    def per_expert(j, _):
        e = wid * 2 + j
        pltpu.sync_copy(lt_hbm.at[e], row)

        _zero_hist(hist, 512)

        def p1(i, se):
            x = row[pl.ds(i * 16, 16)]
            se = se + x
            bits = lax.bitcast_convert_type(x, jnp.int32)
            key = _sortable(bits)
            keys[pl.ds(i * 16, 16)] = key
            bk = key ^ _INT_MIN
            b0 = jnp.right_shift(bk, 23) & np.int32(0x1FF)
            plsc.addupdate_scatter(hist, [b0 * 16 + iota], ones)
            return se

        se = lax.fori_loop(0, _NVREG, p1, jnp.zeros((16,), jnp.float32), unroll=8)
        ssum = jnp.sum(se)

        bin0, g0, c0 = _scan_bins(hist, 512, jnp.int32(_CAP))
        rem = jnp.int32(_CAP) - g0

        def p2(i, carry):
            od, oc = carry
            key = keys[pl.ds(i * 16, 16)]
            idx = i * 16 + iota
            bk = key ^ _INT_MIN
            b0 = jnp.right_shift(bk, 23) & np.int32(0x1FF)
            mhi = b0 > bin0
            meq = b0 == bin0
            chi = plsc.cumsum(mhi.astype(jnp.int32))
            ceq = plsc.cumsum(meq.astype(jnp.int32))
            phi = od + chi - 1
            peq = oc + ceq - 1
            plsc.store_scatter(sk0, [phi], key, mask=mhi)
            plsc.store_scatter(si0, [phi], idx, mask=mhi)
            plsc.store_scatter(ck0, [peq], key, mask=meq)
            plsc.store_scatter(ci0, [peq], idx, mask=meq)
            return (od + plsc.all_reduce_population_count(mhi),
                    oc + plsc.all_reduce_population_count(meq))

        zv = jnp.zeros((16,), jnp.int32)
        od, oc = lax.fori_loop(0, _NVREG, p2, (zv, zv), unroll=4)

        plsc.store_scatter(ck0, [oc + iota], jnp.full((16,), _INT_MIN))
        plsc.store_scatter(ci0, [oc + iota], jnp.full((16,), _IDX_SENT))

        def refine(src_k, src_i, dst_k, dst_i, m, shift, mask, nbins, od, rem):
            _zero_hist(hist, nbins)
            nv = (jnp.minimum(m, _CANDBUF - 16) + 15) // 16

            def hbody(i, _):
                key = src_k[pl.ds(i * 16, 16)]
                bk = key ^ _INT_MIN
                b = jnp.right_shift(bk, shift) & np.int32(mask)
                plsc.addupdate_scatter(hist, [b * 16 + iota], ones)
                return 0

            lax.fori_loop(0, nv, hbody, 0)
            bl, gl, cl = _scan_bins(hist, nbins, rem)

            def cbody(i, carry):
                odc, occ = carry
                key = src_k[pl.ds(i * 16, 16)]
                idx = src_i[pl.ds(i * 16, 16)]
                bk = key ^ _INT_MIN
                b = jnp.right_shift(bk, shift) & np.int32(mask)
                mhi = b > bl
                meq = b == bl
                chi = plsc.cumsum(mhi.astype(jnp.int32))
                ceq = plsc.cumsum(meq.astype(jnp.int32))
                phi = odc + chi - 1
                peq = occ + ceq - 1
                plsc.store_scatter(sk0, [phi], key, mask=mhi)
                plsc.store_scatter(si0, [phi], idx, mask=mhi)
                plsc.store_scatter(dst_k, [peq], key, mask=meq)
                plsc.store_scatter(dst_i, [peq], idx, mask=meq)
                return (odc + plsc.all_reduce_population_count(mhi),
                        occ + plsc.all_reduce_population_count(meq))

            od2, oc2 = lax.fori_loop(0, nv, cbody, (od, jnp.zeros((16,), jnp.int32)))
            plsc.store_scatter(dst_k, [oc2 + iota], jnp.full((16,), _INT_MIN))
            plsc.store_scatter(dst_i, [oc2 + iota], jnp.full((16,), _IDX_SENT))
            return cl, od2, rem - gl

        m, od, rem = refine(ck0, ci0, ck1, ci1, c0, 15, 0xFF, 256, od, rem)
        m, od, rem = refine(ck1, ci1, ck0, ci0, m, 7, 0xFF, 256, od, rem)
        m, od, rem = refine(ck0, ci0, ck1, ci1, m, 0, 0x7F, 128, od, rem)

        nv_t = (jnp.minimum(m, _CANDBUF - 16) + 15) // 16

        def tbody(i, _):
            lanepos = i * 16 + iota
            tm = lanepos < rem
            key = ck1[pl.ds(i * 16, 16)]
            idx = ci1[pl.ds(i * 16, 16)]
            plsc.store_scatter(sk0, [od + lanepos], key, mask=tm)
            plsc.store_scatter(si0, [od + lanepos], idx, mask=tm)
            return 0

        lax.fori_loop(0, nv_t, tbody, 0)

        bufs = ((sk0, si0), (sk1, si1))
        parity = 0
        ksz = 2
        while ksz <= _CAP:
            jj = ksz // 2
            while jj >= 1:
                src_k, src_i = bufs[parity]
                dst_k, dst_i = bufs[1 - parity]

                def stage(v, _, jj=jj, ksz=ksz, src_k=src_k, src_i=src_i,
                          dst_k=dst_k, dst_i=dst_i):
                    gi = v * 16 + iota
                    ks = src_k[pl.ds(v * 16, 16)]
                    is_ = src_i[pl.ds(v * 16, 16)]
                    if jj >= 16:
                        pb = (v * 16) ^ jj
                        kp = src_k[pl.ds(pb, 16)]
                        ip = src_i[pl.ds(pb, 16)]
                    else:
                        pi = gi ^ jj
                        kp = plsc.load_gather(src_k, [pi])
                        ip = plsc.load_gather(src_i, [pi])
                    d_desc = (gi & ksz) == 0
                    is_lower = (gi & jj) == 0
                    sel = d_desc == is_lower
                    cmp = (ks > kp) | ((ks == kp) & (is_ < ip))
                    keep = cmp == sel
                    dst_k[pl.ds(v * 16, 16)] = jnp.where(keep, ks, kp)
                    dst_i[pl.ds(v * 16, 16)] = jnp.where(keep, is_, ip)
                    return 0

                lax.fori_loop(0, _CAP // 16, stage, 0, unroll=4)
                parity = 1 - parity
                jj //= 2
            ksz *= 2

        fin_k, fin_i = bufs[parity]

        inv_v = jnp.full((16,), 1.0, jnp.float32) / lax.broadcast_in_dim(
            ssum, (16,), ())

        def obody(v, _):
            k = fin_k[pl.ds(v * 16, 16)]
            bits = _sortable(k)
            x = lax.bitcast_convert_type(bits, jnp.float32)
            ow[pl.ds(v * 16, 16)] = x * inv_v
            return 0

        lax.fori_loop(0, _CAP // 16, obody, 0)

        pltpu.sync_copy(ow, ow_hbm.at[e])
        pltpu.sync_copy(fin_i, oi_hbm.at[e])
        return 0

    lax.fori_loop(0, 2, per_expert, 0)


def _topk_sc(logits_t):
    mesh = plsc.VectorSubcoreMesh(core_axis_name="c", subcore_axis_name="s")
    fn = functools.partial(
        pl.kernel,
        mesh=mesh,
        compiler_params=pltpu.CompilerParams(needs_layout_passes=False),
        out_type=[
            jax.ShapeDtypeStruct((_E, _CAP), jnp.float32),
            jax.ShapeDtypeStruct((_E, _CAP), jnp.int32),
        ],
        scratch_types=[
            pltpu.VMEM((_N,), jnp.float32),
            pltpu.VMEM((_N,), jnp.int32),
            pltpu.VMEM((512 * 16,), jnp.int32),
            pltpu.VMEM((_CANDBUF,), jnp.int32),
            pltpu.VMEM((_CANDBUF,), jnp.int32),
            pltpu.VMEM((_CANDBUF,), jnp.int32),
            pltpu.VMEM((_CANDBUF,), jnp.int32),
            pltpu.VMEM((_CAP,), jnp.int32),
            pltpu.VMEM((_CAP,), jnp.int32),
            pltpu.VMEM((_CAP,), jnp.int32),
            pltpu.VMEM((_CAP,), jnp.int32),
            pltpu.VMEM((_CAP,), jnp.float32),
        ],
    )(_topk_body)
    return fn(logits_t)


def kernel(hidden_states, W):
    b, s, h = hidden_states.shape
    x_flat = hidden_states.reshape(-1, h)
    p_t = _p_rows(x_flat, W)
    w_t, i_t = _topk_sc(p_t)
    return w_t.T.astype(hidden_states.dtype), i_t.T


# speedup vs baseline: 3.0936x; 1.0524x over previous
import functools

import jax
import jax.numpy as jnp
import numpy as np
from jax import lax
from jax.experimental import pallas as pl
from jax.experimental.pallas import tpu as pltpu
from jax.experimental.pallas import tpu_sc as plsc

_H = 4096
_E = 64
_N = 16384
_CAP = 512
_TB = 512

_NVREG = _N // 16
_CANDBUF = 4112
_INT_MIN = np.int32(-2147483648)
_IDX_SENT = np.int32(2147483647)


def _logits_body(x_ref, w_ref, p_ref, acc_ref):
    i = pl.program_id(0)
    nblk = _N // _TB
    r = lax.dot_general(
        x_ref[...], w_ref[...],
        dimension_numbers=(((1,), (1,)), ((), ())),
        preferred_element_type=jnp.float32,
    )
    acc_ref[i] = r.T

    @pl.when(i == nblk - 1)
    def _():
        m = jnp.max(acc_ref[0], axis=1, keepdims=True)
        for k in range(1, nblk):
            m = jnp.maximum(m, jnp.max(acc_ref[k], axis=1, keepdims=True))
        for k in range(nblk):
            p_ref[:, k * _TB:(k + 1) * _TB] = jnp.exp(acc_ref[k] - m)


def _p_rows(x_flat, w):
    grid = _N // _TB
    return pl.pallas_call(
        _logits_body,
        grid=(grid,),
        in_specs=[
            pl.BlockSpec((_TB, _H), lambda i: (i, 0)),
            pl.BlockSpec((_E, _H), lambda i: (0, 0)),
        ],
        out_specs=pl.BlockSpec((_E, _N), lambda i: (0, 0)),
        out_shape=jax.ShapeDtypeStruct((_E, _N), jnp.float32),
        scratch_shapes=[pltpu.VMEM((_N // _TB, _E, _TB), jnp.float32)],
    )(x_flat, w)


def _iota16():
    return lax.iota(jnp.int32, 16)


def _sortable(bits):
    return bits ^ (jnp.right_shift(bits, 31) & np.int32(0x7FFFFFFF))


def _scan_bins(hist, start, target):
    def read(b):
        return jnp.sum(hist[pl.ds(b * 16, 16)])

    def cond(state):
        b, g, c = state
        return (g + c) < target

    def body(state):
        b, g, c = state
        b2 = b - 1
        return (b2, g + c, read(b2))

    return lax.while_loop(cond, body, (start, jnp.int32(0), read(start)))


def _zero_hist(hist, nbins):
    zeros = jnp.zeros((16,), jnp.int32)

    def zbody(z, _):
        hist[pl.ds(z * 16, 16)] = zeros
        return 0

    lax.fori_loop(0, nbins, zbody, 0, unroll=8)


def _topk_body(lt_hbm, ow_hbm, oi_hbm,
               row, keys, hist, ck0, ci0, ck1, ci1, sk0, si0, sk1, si1, ow):
    info = plsc.get_sparse_core_info()
    nc = info.num_cores
    wid = lax.axis_index("s") * nc + lax.axis_index("c")
    iota = _iota16()
    ones = jnp.ones((16,), jnp.int32)

    def per_expert(j, _):
        e = wid * 2 + j
        pltpu.sync_copy(lt_hbm.at[e], row)

        _zero_hist(hist, 256)

        def p1(i, carry):
            se, mk = carry
            x = row[pl.ds(i * 16, 16)]
            se = se + x
            bits = lax.bitcast_convert_type(x, jnp.int32)
            key = _sortable(bits)
            keys[pl.ds(i * 16, 16)] = key
            mk = jnp.maximum(mk, key)
            bk = key ^ _INT_MIN
            b0 = jnp.right_shift(bk, 24) & np.int32(0xFF)
            plsc.addupdate_scatter(hist, [b0 * 16 + iota], ones)
            return se, mk

        se, mk = lax.fori_loop(
            0, _NVREG, p1,
            (jnp.zeros((16,), jnp.float32), jnp.full((16,), _INT_MIN)),
            unroll=8)
        ssum = jnp.sum(se)
        maxbin = jnp.right_shift(jnp.max(mk) ^ _INT_MIN, 24) & np.int32(0xFF)

        bin0, g0, c0 = _scan_bins(hist, maxbin, jnp.int32(_CAP))
        rem = jnp.int32(_CAP) - g0

        def p2(i, carry):
            od, oc = carry
            key = keys[pl.ds(i * 16, 16)]
            idx = i * 16 + iota
            bk = key ^ _INT_MIN
            b0 = jnp.right_shift(bk, 24) & np.int32(0xFF)
            mhi = b0 > bin0
            meq = b0 == bin0
            chi = plsc.cumsum(mhi.astype(jnp.int32))
            ceq = plsc.cumsum(meq.astype(jnp.int32))
            phi = od + chi - 1
            peq = oc + ceq - 1
            plsc.store_scatter(sk0, [phi], key, mask=mhi)
            plsc.store_scatter(si0, [phi], idx, mask=mhi)
            plsc.store_scatter(ck0, [peq], key, mask=meq)
            plsc.store_scatter(ci0, [peq], idx, mask=meq)
            return (od + plsc.all_reduce_population_count(mhi),
                    oc + plsc.all_reduce_population_count(meq))

        zv = jnp.zeros((16,), jnp.int32)
        od, oc = lax.fori_loop(0, _NVREG, p2, (zv, zv), unroll=4)

        plsc.store_scatter(ck0, [oc + iota], jnp.full((16,), _INT_MIN))
        plsc.store_scatter(ci0, [oc + iota], jnp.full((16,), _IDX_SENT))

        def refine(src_k, src_i, dst_k, dst_i, m, shift, mask, nbins, od, rem):
            _zero_hist(hist, nbins)
            nv = (jnp.minimum(m, _CANDBUF - 16) + 15) // 16

            def hbody(i, _):
                key = src_k[pl.ds(i * 16, 16)]
                bk = key ^ _INT_MIN
                b = jnp.right_shift(bk, shift) & np.int32(mask)
                plsc.addupdate_scatter(hist, [b * 16 + iota], ones)
                return 0

            lax.fori_loop(0, nv, hbody, 0)
            bl, gl, cl = _scan_bins(hist, jnp.int32(nbins - 1), rem)

            def cbody(i, carry):
                odc, occ = carry
                key = src_k[pl.ds(i * 16, 16)]
                idx = src_i[pl.ds(i * 16, 16)]
                bk = key ^ _INT_MIN
                b = jnp.right_shift(bk, shift) & np.int32(mask)
                mhi = b > bl
                meq = b == bl
                chi = plsc.cumsum(mhi.astype(jnp.int32))
                ceq = plsc.cumsum(meq.astype(jnp.int32))
                phi = odc + chi - 1
                peq = occ + ceq - 1
                plsc.store_scatter(sk0, [phi], key, mask=mhi)
                plsc.store_scatter(si0, [phi], idx, mask=mhi)
                plsc.store_scatter(dst_k, [peq], key, mask=meq)
                plsc.store_scatter(dst_i, [peq], idx, mask=meq)
                return (odc + plsc.all_reduce_population_count(mhi),
                        occ + plsc.all_reduce_population_count(meq))

            od2, oc2 = lax.fori_loop(0, nv, cbody, (od, jnp.zeros((16,), jnp.int32)))
            plsc.store_scatter(dst_k, [oc2 + iota], jnp.full((16,), _INT_MIN))
            plsc.store_scatter(dst_i, [oc2 + iota], jnp.full((16,), _IDX_SENT))
            return cl, od2, rem - gl

        m, od, rem = refine(ck0, ci0, ck1, ci1, c0, 16, 0xFF, 256, od, rem)
        m, od, rem = refine(ck1, ci1, ck0, ci0, m, 8, 0xFF, 256, od, rem)
        m, od, rem = refine(ck0, ci0, ck1, ci1, m, 0, 0xFF, 256, od, rem)

        nv_t = (jnp.minimum(m, _CANDBUF - 16) + 15) // 16

        def tbody(i, _):
            lanepos = i * 16 + iota
            tm = lanepos < rem
            key = ck1[pl.ds(i * 16, 16)]
            idx = ci1[pl.ds(i * 16, 16)]
            plsc.store_scatter(sk0, [od + lanepos], key, mask=tm)
            plsc.store_scatter(si0, [od + lanepos], idx, mask=tm)
            return 0

        lax.fori_loop(0, nv_t, tbody, 0)

        bufs = ((sk0, si0), (sk1, si1))
        parity = 0
        ksz = 2
        while ksz <= _CAP:
            jj = ksz // 2
            while jj >= 1:
                src_k, src_i = bufs[parity]
                dst_k, dst_i = bufs[1 - parity]

                def stage(v, _, jj=jj, ksz=ksz, src_k=src_k, src_i=src_i,
                          dst_k=dst_k, dst_i=dst_i):
                    gi = v * 16 + iota
                    ks = src_k[pl.ds(v * 16, 16)]
                    is_ = src_i[pl.ds(v * 16, 16)]
                    if jj >= 16:
                        pb = (v * 16) ^ jj
                        kp = src_k[pl.ds(pb, 16)]
                        ip = src_i[pl.ds(pb, 16)]
                    else:
                        pi = gi ^ jj
                        kp = plsc.load_gather(src_k, [pi])
                        ip = plsc.load_gather(src_i, [pi])
                    d_desc = (gi & ksz) == 0
                    is_lower = (gi & jj) == 0
                    sel = d_desc == is_lower
                    cmp = (ks > kp) | ((ks == kp) & (is_ < ip))
                    keep = cmp == sel
                    dst_k[pl.ds(v * 16, 16)] = jnp.where(keep, ks, kp)
                    dst_i[pl.ds(v * 16, 16)] = jnp.where(keep, is_, ip)
                    return 0

                lax.fori_loop(0, _CAP // 16, stage, 0, unroll=4)
                parity = 1 - parity
                jj //= 2
            ksz *= 2

        fin_k, fin_i = bufs[parity]

        inv_v = jnp.full((16,), 1.0, jnp.float32) / lax.broadcast_in_dim(
            ssum, (16,), ())

        def obody(v, _):
            k = fin_k[pl.ds(v * 16, 16)]
            bits = _sortable(k)
            x = lax.bitcast_convert_type(bits, jnp.float32)
            ow[pl.ds(v * 16, 16)] = x * inv_v
            return 0

        lax.fori_loop(0, _CAP // 16, obody, 0)

        pltpu.sync_copy(ow, ow_hbm.at[e])
        pltpu.sync_copy(fin_i, oi_hbm.at[e])
        return 0

    lax.fori_loop(0, 2, per_expert, 0)


def _topk_sc(logits_t):
    mesh = plsc.VectorSubcoreMesh(core_axis_name="c", subcore_axis_name="s")
    fn = functools.partial(
        pl.kernel,
        mesh=mesh,
        compiler_params=pltpu.CompilerParams(needs_layout_passes=False),
        out_type=[
            jax.ShapeDtypeStruct((_E, _CAP), jnp.float32),
            jax.ShapeDtypeStruct((_E, _CAP), jnp.int32),
        ],
        scratch_types=[
            pltpu.VMEM((_N,), jnp.float32),
            pltpu.VMEM((_N,), jnp.int32),
            pltpu.VMEM((512 * 16,), jnp.int32),
            pltpu.VMEM((_CANDBUF,), jnp.int32),
            pltpu.VMEM((_CANDBUF,), jnp.int32),
            pltpu.VMEM((_CANDBUF,), jnp.int32),
            pltpu.VMEM((_CANDBUF,), jnp.int32),
            pltpu.VMEM((_CAP,), jnp.int32),
            pltpu.VMEM((_CAP,), jnp.int32),
            pltpu.VMEM((_CAP,), jnp.int32),
            pltpu.VMEM((_CAP,), jnp.int32),
            pltpu.VMEM((_CAP,), jnp.float32),
        ],
    )(_topk_body)
    return fn(logits_t)


def kernel(hidden_states, W):
    b, s, h = hidden_states.shape
    x_flat = hidden_states.reshape(-1, h)
    p_t = _p_rows(x_flat, W)
    w_t, i_t = _topk_sc(p_t)
    return w_t.T.astype(hidden_states.dtype), i_t.T
